# baseline (speedup 1.0000x reference)
import jax
import jax.numpy as jnp
import numpy as np
from jax import lax
from jax.experimental import pallas as pl
from jax.experimental.pallas import tpu as pltpu
from jax.experimental.pallas import tpu_sc as plsc

_NUM_WORDS = 100000
_NUM_BUCKETS = 1 << 20
_MASK = _NUM_BUCKETS - 1
_K = 3
_D = 64
_DP = 128
_B = 16384
_C = 5

_rs = np.random.RandomState(1139)
_HA = tuple(int(x) for x in _rs.randint(1, 21000, size=(_K,)))
_HB = tuple(int(x) for x in _rs.randint(0, _NUM_BUCKETS, size=(_K,)))

_NC, _NS, _L = 2, 16, 16
_NWK = _NC * _NS
_BPW = _B // _NWK
_CB = 32
_NCH = _BPW // _CB
_R = _C * _K

_HBLK = 128
_NHB = _NUM_BUCKETS // _HBLK
_HBW = _NHB // _NWK
_MB = 2
_NM = _HBW // _MB
_O = _D // 8


def _hash(ids, k):
    return (((ids * _HA[k]) & _MASK) + _HB[k]) & _MASK


_TCW = 1024


def _tc_fmt_body(tT_ref, tC_ref, out_ref):
    eye = jnp.eye(_D, dtype=jnp.float32)
    dn = (((0,), (0,)), ((), ()))
    out_ref[:, : _D] = lax.dot_general(tT_ref[...], eye, dn,
                                       preferred_element_type=jnp.float32)
    out_ref[:, _D:] = lax.dot_general(tC_ref[...], eye, dn,
                                      preferred_element_type=jnp.float32)


def _fmt_body(tT4, tC4, tab_hbm,
              bT0, bC0, bT1, bC1, obA, obB, lsem0, lsem1, wsemA, wsemB):
    wid = lax.axis_index("s") * _NC + lax.axis_index("c")
    hb_base = wid * _HBW

    def fire(mi, bT, bC, sem):
        hb0 = hb_base + mi * _MB
        for o in range(_O):
            pltpu.async_copy(tT4.at[o, pl.ds(hb0, _MB)], bT.at[o], sem)
            pltpu.async_copy(tC4.at[o, pl.ds(hb0, _MB)], bC.at[o], sem)

    def wait_loads(bT, bC, sem):
        for o in range(_O):
            pltpu.make_async_copy(tT4.at[0, pl.ds(0, _MB)], bT.at[o], sem).wait()
            pltpu.make_async_copy(tC4.at[0, pl.ds(0, _MB)], bC.at[o], sem).wait()

    lanes = lax.iota(jnp.int32, _L)
    rhi = [((lanes + l * _L) // 8) for l in range(_HBLK // _L)]
    rlo = [((lanes + l * _L) % 8) for l in range(_HBLK // _L)]

    def transpose(bT, bC, m, ob):
        @plsc.parallel_loop(0, _D, 1, unroll=2)
        def dstep(d):
            o = d // 8
            r = d % 8
            dT = jnp.full((_L,), d, jnp.int32)
            dC = dT + _D
            for hh in range(_HBLK // _L):
                plsc.store_scatter(ob, [rhi[hh], rlo[hh], dT],
                                   bT[o, m, r, pl.ds(hh * _L, _L)])
                plsc.store_scatter(ob, [rhi[hh], rlo[hh], dC],
                                   bC[o, m, r, pl.ds(hh * _L, _L)])

    def wrt(mi, m, ob, wsem):
        t0 = (hb_base + mi * _MB + m) * (_HBLK // 8)
        pltpu.async_copy(ob, tab_hbm.at[pl.ds(t0, _HBLK // 8)], wsem)

    def wait_wrt(ob, wsem):
        pltpu.make_async_copy(ob, tab_hbm.at[pl.ds(0, _HBLK // 8)], wsem).wait()

    fire(0, bT0, bC0, lsem0)

    def blk(i, carry):
        m0 = 2 * i
        fire(jnp.minimum(m0 + 1, _NM - 1), bT1, bC1, lsem1)
        wait_loads(bT0, bC0, lsem0)

        @pl.when(i > 0)
        def _():
            wait_wrt(obA, wsemA)
            wait_wrt(obB, wsemB)

        transpose(bT0, bC0, 0, obA)
        wrt(m0, 0, obA, wsemA)
        transpose(bT0, bC0, 1, obB)
        wrt(m0, 1, obB, wsemB)
        fire(jnp.minimum(m0 + 2, _NM - 1), bT0, bC0, lsem0)
        wait_loads(bT1, bC1, lsem1)
        wait_wrt(obA, wsemA)
        wait_wrt(obB, wsemB)
        transpose(bT1, bC1, 0, obA)
        wrt(m0 + 1, 0, obA, wsemA)
        transpose(bT1, bC1, 1, obB)
        wrt(m0 + 1, 1, obB, wsemB)
        return carry

    lax.fori_loop(0, _NM // 2, blk, 0)
    wait_loads(bT0, bC0, lsem0)
    wait_wrt(obA, wsemA)
    wait_wrt(obB, wsemB)


def _body(tgt_hbm, ctx_hbm, tab_hbm, imp_hbm, out_hbm,
          tid_v, cid_v, idxT, idxC, wTr, wCr_v, rowsT, rowsC, out_v, sem):
    wid = lax.axis_index("s") * _NC + lax.axis_index("c")

    def chunk(ch, carry):
        base = wid * _BPW + ch * _CB
        pltpu.sync_copy(tgt_hbm.at[pl.ds(base, _CB)], tid_v)
        pltpu.sync_copy(ctx_hbm.at[pl.ds(base * _C, _CB * _C)], cid_v)
        for i in range(_CB // _L):
            lanes = lax.iota(jnp.int32, _L) + i * _L
            ids = tid_v[pl.ds(i * _L, _L)]
            for k in range(_K):
                idxT[k, pl.ds(i * _L, _L)] = _hash(ids, k)
            for c in range(_C):
                cids = plsc.load_gather(cid_v, [lanes * _C + c])
                idxC[_R + c, pl.ds(i * _L, _L)] = cids
                for k in range(_K):
                    idxC[c * _K + k, pl.ds(i * _L, _L)] = _hash(cids, k)
        cps = [pltpu.async_copy(imp_hbm.at[tid_v], wTr, sem)]
        for k in range(_K):
            cps.append(pltpu.async_copy(tab_hbm.at[idxT.at[k]], rowsT.at[k], sem))
        for r in range(_R):
            cps.append(pltpu.async_copy(tab_hbm.at[idxC.at[r]], rowsC.at[r], sem))
        for c in range(_C):
            cps.append(pltpu.async_copy(imp_hbm.at[idxC.at[_R + c]], wCr_v.at[c], sem))
        for cp in cps:
            cp.wait()
        for g in range(_CB // _L):
            b0 = g * _L
            blane = lax.iota(jnp.int32, _L) + b0
            ksp = [jnp.full((_L,), k, jnp.int32) for k in range(_K)]
            kcs = [jnp.full((_L,), _K + k, jnp.int32) for k in range(_K)]
            rsp = [jnp.full((_L,), r, jnp.int32) for r in range(_R)]
            csp = [jnp.full((_L,), c, jnp.int32) for c in range(_C)]
            wTk = [plsc.load_gather(wTr, [blane, ksp[k]]) for k in range(_K)]
            wCk = [plsc.load_gather(wCr_v, [csp[r // _K], blane, kcs[r % _K]])
                   for r in range(_R)]

            def dbody(d, accs):
                dsp = jnp.full((_L,), d, jnp.int32)
                dspc = jnp.full((_L,), d + _D, jnp.int32)
                we = wTk[0] * plsc.load_gather(rowsT, [ksp[0], blane, dsp])
                for k in range(1, _K):
                    we = we + wTk[k] * plsc.load_gather(rowsT, [ksp[k], blane, dsp])
                out = []
                for c in range(_C):
                    r0 = c * _K
                    ce = wCk[r0] * plsc.load_gather(rowsC, [rsp[r0], blane, dspc])
                    for k in range(1, _K):
                        r = r0 + k
                        ce = ce + wCk[r] * plsc.load_gather(rowsC, [rsp[r], blane, dspc])
                    out.append(accs[c] + we * ce)
                return tuple(out)

            accs = plsc.parallel_loop(
                0, _D, 1, unroll=4,
                carry=tuple(jnp.zeros((_L,), jnp.float32) for _ in range(_C)),
            )(dbody)
            for c in range(_C):
                plsc.store_scatter(out_v, [blane * _C + c], accs[c])
        pltpu.sync_copy(out_v, out_hbm.at[pl.ds(base * _C, _CB * _C)])
        return carry

    lax.fori_loop(0, _NCH, chunk, 0)


def kernel(target, context, table_target, imp_target, table_context, imp_context):
    tgt = target.reshape(_B).astype(jnp.int32)
    ctx = context.reshape(_B * _C).astype(jnp.int32)
    imp = jnp.pad(jnp.concatenate([imp_target, imp_context], axis=1),
                  ((0, 0), (0, _DP - 2 * _K)))
    mesh = plsc.VectorSubcoreMesh(core_axis_name="c", subcore_axis_name="s",
                                  num_cores=_NC, num_subcores=_NS)
    cparams = pltpu.CompilerParams(needs_layout_passes=False,
                                   use_tc_tiling_on_sc=True)

    tab = pl.pallas_call(
        _tc_fmt_body,
        grid=(_NUM_BUCKETS // _TCW,),
        in_specs=[
            pl.BlockSpec((_D, _TCW), lambda j: (0, j)),
            pl.BlockSpec((_D, _TCW), lambda j: (0, j)),
        ],
        out_specs=pl.BlockSpec((_TCW, _DP), lambda j: (j, 0)),
        out_shape=jax.ShapeDtypeStruct((_NUM_BUCKETS, _DP), jnp.float32),
    )(table_target.T, table_context.T)

    out = pl.kernel(
        _body,
        out_type=jax.ShapeDtypeStruct((_B * _C,), jnp.float32),
        mesh=mesh,
        compiler_params=cparams,
        scratch_types=[
            pltpu.VMEM((_CB,), jnp.int32),
            pltpu.VMEM((_CB * _C,), jnp.int32),
            pltpu.VMEM((_K, _CB), jnp.int32),
            pltpu.VMEM((_R + _C, _CB), jnp.int32),
            pltpu.VMEM((_CB, _DP), jnp.float32),
            pltpu.VMEM((_C, _CB, _DP), jnp.float32),
            pltpu.VMEM((_K, _CB, _DP), jnp.float32),
            pltpu.VMEM((_R, _CB, _DP), jnp.float32),
            pltpu.VMEM((_CB * _C,), jnp.float32),
            pltpu.SemaphoreType.DMA,
        ],
    )(tgt, ctx, tab, imp)
    return out.reshape(_B, _C)


# speedup vs baseline: 1.4018x; 1.4018x over previous
import jax
import jax.numpy as jnp
import numpy as np
from jax import lax
from jax.experimental import pallas as pl
from jax.experimental.pallas import tpu as pltpu
from jax.experimental.pallas import tpu_sc as plsc

_NUM_WORDS = 100000
_NUM_BUCKETS = 1 << 20
_MASK = _NUM_BUCKETS - 1
_K = 3
_D = 64
_DP = 128
_B = 16384
_C = 5

_rs = np.random.RandomState(1139)
_HA = tuple(int(x) for x in _rs.randint(1, 21000, size=(_K,)))
_HB = tuple(int(x) for x in _rs.randint(0, _NUM_BUCKETS, size=(_K,)))

_NC, _NS, _L = 2, 16, 16
_NWK = _NC * _NS
_BPW = _B // _NWK
_CB = 32
_NCH = _BPW // _CB
_R = _C * _K

_HBLK = 128
_NHB = _NUM_BUCKETS // _HBLK
_HBW = _NHB // _NWK
_MB = 2
_NM = _HBW // _MB
_O = _D // 8


def _hash(ids, k):
    return (((ids * _HA[k]) & _MASK) + _HB[k]) & _MASK


_TCW = 4096


def _tc_fmt_body(tT_ref, tC_ref, out_ref):
    out_ref[:, : _D] = tT_ref[...].T
    out_ref[:, _D:] = tC_ref[...].T


def _fmt_body(tT4, tC4, tab_hbm,
              bT0, bC0, bT1, bC1, obA, obB, lsem0, lsem1, wsemA, wsemB):
    wid = lax.axis_index("s") * _NC + lax.axis_index("c")
    hb_base = wid * _HBW

    def fire(mi, bT, bC, sem):
        hb0 = hb_base + mi * _MB
        for o in range(_O):
            pltpu.async_copy(tT4.at[o, pl.ds(hb0, _MB)], bT.at[o], sem)
            pltpu.async_copy(tC4.at[o, pl.ds(hb0, _MB)], bC.at[o], sem)

    def wait_loads(bT, bC, sem):
        for o in range(_O):
            pltpu.make_async_copy(tT4.at[0, pl.ds(0, _MB)], bT.at[o], sem).wait()
            pltpu.make_async_copy(tC4.at[0, pl.ds(0, _MB)], bC.at[o], sem).wait()

    lanes = lax.iota(jnp.int32, _L)
    rhi = [((lanes + l * _L) // 8) for l in range(_HBLK // _L)]
    rlo = [((lanes + l * _L) % 8) for l in range(_HBLK // _L)]

    def transpose(bT, bC, m, ob):
        @plsc.parallel_loop(0, _D, 1, unroll=2)
        def dstep(d):
            o = d // 8
            r = d % 8
            dT = jnp.full((_L,), d, jnp.int32)
            dC = dT + _D
            for hh in range(_HBLK // _L):
                plsc.store_scatter(ob, [rhi[hh], rlo[hh], dT],
                                   bT[o, m, r, pl.ds(hh * _L, _L)])
                plsc.store_scatter(ob, [rhi[hh], rlo[hh], dC],
                                   bC[o, m, r, pl.ds(hh * _L, _L)])

    def wrt(mi, m, ob, wsem):
        t0 = (hb_base + mi * _MB + m) * (_HBLK // 8)
        pltpu.async_copy(ob, tab_hbm.at[pl.ds(t0, _HBLK // 8)], wsem)

    def wait_wrt(ob, wsem):
        pltpu.make_async_copy(ob, tab_hbm.at[pl.ds(0, _HBLK // 8)], wsem).wait()

    fire(0, bT0, bC0, lsem0)

    def blk(i, carry):
        m0 = 2 * i
        fire(jnp.minimum(m0 + 1, _NM - 1), bT1, bC1, lsem1)
        wait_loads(bT0, bC0, lsem0)

        @pl.when(i > 0)
        def _():
            wait_wrt(obA, wsemA)
            wait_wrt(obB, wsemB)

        transpose(bT0, bC0, 0, obA)
        wrt(m0, 0, obA, wsemA)
        transpose(bT0, bC0, 1, obB)
        wrt(m0, 1, obB, wsemB)
        fire(jnp.minimum(m0 + 2, _NM - 1), bT0, bC0, lsem0)
        wait_loads(bT1, bC1, lsem1)
        wait_wrt(obA, wsemA)
        wait_wrt(obB, wsemB)
        transpose(bT1, bC1, 0, obA)
        wrt(m0 + 1, 0, obA, wsemA)
        transpose(bT1, bC1, 1, obB)
        wrt(m0 + 1, 1, obB, wsemB)
        return carry

    lax.fori_loop(0, _NM // 2, blk, 0)
    wait_loads(bT0, bC0, lsem0)
    wait_wrt(obA, wsemA)
    wait_wrt(obB, wsemB)


def _body(tgt_hbm, ctx_hbm, tab_hbm, imp_hbm, out_hbm,
          tid_v, cid_v, idxT, idxC, wTr, wCr_v, rowsT, rowsC, out_v, sem):
    wid = lax.axis_index("s") * _NC + lax.axis_index("c")

    def chunk(ch, carry):
        base = wid * _BPW + ch * _CB
        pltpu.sync_copy(tgt_hbm.at[pl.ds(base, _CB)], tid_v)
        pltpu.sync_copy(ctx_hbm.at[pl.ds(base * _C, _CB * _C)], cid_v)
        for i in range(_CB // _L):
            lanes = lax.iota(jnp.int32, _L) + i * _L
            ids = tid_v[pl.ds(i * _L, _L)]
            for k in range(_K):
                idxT[k, pl.ds(i * _L, _L)] = _hash(ids, k)
            for c in range(_C):
                cids = plsc.load_gather(cid_v, [lanes * _C + c])
                idxC[_R + c, pl.ds(i * _L, _L)] = cids
                for k in range(_K):
                    idxC[c * _K + k, pl.ds(i * _L, _L)] = _hash(cids, k)
        cps = [pltpu.async_copy(imp_hbm.at[tid_v], wTr, sem)]
        for k in range(_K):
            cps.append(pltpu.async_copy(tab_hbm.at[idxT.at[k]], rowsT.at[k], sem))
        for r in range(_R):
            cps.append(pltpu.async_copy(tab_hbm.at[idxC.at[r]], rowsC.at[r], sem))
        for c in range(_C):
            cps.append(pltpu.async_copy(imp_hbm.at[idxC.at[_R + c]], wCr_v.at[c], sem))
        for cp in cps:
            cp.wait()
        for g in range(_CB // _L):
            b0 = g * _L
            blane = lax.iota(jnp.int32, _L) + b0
            ksp = [jnp.full((_L,), k, jnp.int32) for k in range(_K)]
            kcs = [jnp.full((_L,), _K + k, jnp.int32) for k in range(_K)]
            rsp = [jnp.full((_L,), r, jnp.int32) for r in range(_R)]
            csp = [jnp.full((_L,), c, jnp.int32) for c in range(_C)]
            wTk = [plsc.load_gather(wTr, [blane, ksp[k]]) for k in range(_K)]
            wCk = [plsc.load_gather(wCr_v, [csp[r // _K], blane, kcs[r % _K]])
                   for r in range(_R)]

            def dbody(d, accs):
                dsp = jnp.full((_L,), d, jnp.int32)
                dspc = jnp.full((_L,), d + _D, jnp.int32)
                we = wTk[0] * plsc.load_gather(rowsT, [ksp[0], blane, dsp])
                for k in range(1, _K):
                    we = we + wTk[k] * plsc.load_gather(rowsT, [ksp[k], blane, dsp])
                out = []
                for c in range(_C):
                    r0 = c * _K
                    ce = wCk[r0] * plsc.load_gather(rowsC, [rsp[r0], blane, dspc])
                    for k in range(1, _K):
                        r = r0 + k
                        ce = ce + wCk[r] * plsc.load_gather(rowsC, [rsp[r], blane, dspc])
                    out.append(accs[c] + we * ce)
                return tuple(out)

            accs = plsc.parallel_loop(
                0, _D, 1, unroll=4,
                carry=tuple(jnp.zeros((_L,), jnp.float32) for _ in range(_C)),
            )(dbody)
            for c in range(_C):
                plsc.store_scatter(out_v, [blane * _C + c], accs[c])
        pltpu.sync_copy(out_v, out_hbm.at[pl.ds(base * _C, _CB * _C)])
        return carry

    lax.fori_loop(0, _NCH, chunk, 0)


def kernel(target, context, table_target, imp_target, table_context, imp_context):
    tgt = target.reshape(_B).astype(jnp.int32)
    ctx = context.reshape(_B * _C).astype(jnp.int32)
    imp = jnp.pad(jnp.concatenate([imp_target, imp_context], axis=1),
                  ((0, 0), (0, _DP - 2 * _K)))
    mesh = plsc.VectorSubcoreMesh(core_axis_name="c", subcore_axis_name="s",
                                  num_cores=_NC, num_subcores=_NS)
    cparams = pltpu.CompilerParams(needs_layout_passes=False,
                                   use_tc_tiling_on_sc=True)

    tab = pl.pallas_call(
        _tc_fmt_body,
        grid=(_NUM_BUCKETS // _TCW,),
        in_specs=[
            pl.BlockSpec((_D, _TCW), lambda j: (0, j)),
            pl.BlockSpec((_D, _TCW), lambda j: (0, j)),
        ],
        out_specs=pl.BlockSpec((_TCW, _DP), lambda j: (j, 0)),
        out_shape=jax.ShapeDtypeStruct((_NUM_BUCKETS, _DP), jnp.float32),
    )(table_target.T, table_context.T)

    out = pl.kernel(
        _body,
        out_type=jax.ShapeDtypeStruct((_B * _C,), jnp.float32),
        mesh=mesh,
        compiler_params=cparams,
        scratch_types=[
            pltpu.VMEM((_CB,), jnp.int32),
            pltpu.VMEM((_CB * _C,), jnp.int32),
            pltpu.VMEM((_K, _CB), jnp.int32),
            pltpu.VMEM((_R + _C, _CB), jnp.int32),
            pltpu.VMEM((_CB, _DP), jnp.float32),
            pltpu.VMEM((_C, _CB, _DP), jnp.float32),
            pltpu.VMEM((_K, _CB, _DP), jnp.float32),
            pltpu.VMEM((_R, _CB, _DP), jnp.float32),
            pltpu.VMEM((_CB * _C,), jnp.float32),
            pltpu.SemaphoreType.DMA,
        ],
    )(tgt, ctx, tab, imp)
    return out.reshape(_B, _C)
